# baseline (speedup 1.0000x reference)
import functools

import jax
import jax.numpy as jnp
from jax import lax
from jax.experimental import pallas as pl
from jax.experimental.pallas import tpu as pltpu
from jax.experimental.pallas import tpu_sc as plsc

VOCAB = 1000000
EMBED = 64
BATCH = 16384

NC = 2
NS = 16
L = 16
NW = NC * NS
BPW = BATCH // NW
NCHUNK = 4
CH = BPW // NCHUNK

_mesh = plsc.VectorSubcoreMesh(core_axis_name="c", subcore_axis_name="s")


@functools.partial(
    pl.kernel,
    out_type=jax.ShapeDtypeStruct((BATCH,), jnp.float32),
    mesh=_mesh,
    compiler_params=pltpu.CompilerParams(needs_layout_passes=False,
                                         use_tc_tiling_on_sc=False),
    scratch_types=[
        pltpu.VMEM((NCHUNK, CH), jnp.int32),
        pltpu.VMEM((NCHUNK, CH), jnp.int32),
        pltpu.VMEM((BPW, EMBED), jnp.float32),
        pltpu.VMEM((BPW, EMBED), jnp.float32),
        pltpu.VMEM((BPW,), jnp.float32),
        pltpu.VMEM((BPW,), jnp.float32),
        pltpu.VMEM((BPW,), jnp.float32),
        pltpu.SemaphoreType.DMA,
    ],
)
def _sc_kernel(widx_hbm, cidx_hbm, wtab_hbm, ctab_hbm, wb_hbm, cb_hbm,
               out_hbm, widx_v, cidx_v, wrows_v, crows_v, wb_v, cb_v,
               out_v, sem):
    wid = lax.axis_index("s") * NC + lax.axis_index("c")
    base = wid * BPW

    pltpu.sync_copy(widx_hbm.at[wid], widx_v)
    pltpu.sync_copy(cidx_hbm.at[wid], cidx_v)

    copies = []
    for j in range(NCHUNK):
        sl = pl.ds(j * CH, CH)
        copies.append(pltpu.async_copy(wtab_hbm.at[widx_v.at[j]], wrows_v.at[sl], sem))
        copies.append(pltpu.async_copy(ctab_hbm.at[cidx_v.at[j]], crows_v.at[sl], sem))
        copies.append(pltpu.async_copy(wb_hbm.at[widx_v.at[j]], wb_v.at[sl], sem))
        copies.append(pltpu.async_copy(cb_hbm.at[cidx_v.at[j]], cb_v.at[sl], sem))
    for c in copies:
        c.wait()

    lane = lax.iota(jnp.int32, L)

    def group(g, carry):
        out16 = jnp.zeros((L,), jnp.float32)
        for r in range(L):
            b = g * L + r
            acc = wrows_v[b, pl.ds(0, L)] * crows_v[b, pl.ds(0, L)]
            for k in range(1, EMBED // L):
                acc = acc + wrows_v[b, pl.ds(k * L, L)] * crows_v[b, pl.ds(k * L, L)]
            out16 = jnp.where(lane == r, jnp.sum(acc), out16)
        sl = pl.ds(g * L, L)
        out_v[sl] = out16 + wb_v[sl] + cb_v[sl]
        return carry

    lax.fori_loop(0, BPW // L, group, 0)

    pltpu.sync_copy(out_v, out_hbm.at[pl.ds(base, BPW)])


def kernel(word_idx, context_idx, word_embeddings, context_embeddings,
           word_biases, context_biases):
    widx = word_idx.astype(jnp.int32).reshape(NW, NCHUNK, CH)
    cidx = context_idx.astype(jnp.int32).reshape(NW, NCHUNK, CH)
    wb = word_biases.reshape(VOCAB)
    cb = context_biases.reshape(VOCAB)
    return _sc_kernel(widx, cidx, word_embeddings, context_embeddings, wb, cb)
